# Initial kernel scaffold; baseline (speedup 1.0000x reference)
import jax
import jax.numpy as jnp
from jax.experimental import pallas as pl


def kernel(x, plane_idx, W1, b1, W2, b2):
    raise NotImplementedError("write your pallas kernel here")


# speedup vs baseline: 17.7202x; 17.7202x over previous
import functools

import jax
import jax.numpy as jnp
from jax import lax
from jax.experimental import pallas as pl
from jax.experimental.pallas import tpu as pltpu

_B = 256


def _mlp_step(sg_ref, st_ref, off_ref, x_ref, w1_ref, b1_ref, w2_ref, b2_ref,
              o_ref):
    s = pl.program_id(0)
    g = sg_ref[s]
    t = st_ref[s]
    lo = off_ref[g]
    hi = off_ref[g + 1]
    x = x_ref[...]
    h = lax.dot_general(x, w1_ref[0], (((1,), (1,)), ((), ())),
                        preferred_element_type=jnp.float32)
    h = h + b1_ref[0]
    h = 0.5 * h * (1.0 + lax.erf(h * (2.0 ** -0.5)))
    o = lax.dot_general(h, w2_ref[0], (((1,), (1,)), ((), ())),
                        preferred_element_type=jnp.float32)
    o = o + b2_ref[0]
    row = t * _B + lax.broadcasted_iota(jnp.int32, (_B, 1), 0)
    mask = (row >= lo) & (row < hi)
    o_ref[...] = jnp.where(mask, o, o_ref[...])


def _grouped_mlp(x_sorted, offsets, step_g, step_t, W1, b1, W2, b2, *,
                 interpret=False):
    N, D_IN = x_sorted.shape
    Lp, D_HID, _ = W1.shape
    D_OUT = W2.shape[1]
    S = step_g.shape[0]
    grid_spec = pltpu.PrefetchScalarGridSpec(
        num_scalar_prefetch=3,
        grid=(S,),
        in_specs=[
            pl.BlockSpec((_B, D_IN), lambda s, sg, st, off: (st[s], 0)),
            pl.BlockSpec((1, D_HID, D_IN), lambda s, sg, st, off: (sg[s], 0, 0)),
            pl.BlockSpec((1, 1, D_HID), lambda s, sg, st, off: (sg[s], 0, 0)),
            pl.BlockSpec((1, D_OUT, D_HID), lambda s, sg, st, off: (sg[s], 0, 0)),
            pl.BlockSpec((1, 1, D_OUT), lambda s, sg, st, off: (sg[s], 0, 0)),
        ],
        out_specs=pl.BlockSpec((_B, D_OUT), lambda s, sg, st, off: (st[s], 0)),
    )
    return pl.pallas_call(
        _mlp_step,
        grid_spec=grid_spec,
        out_shape=jax.ShapeDtypeStruct((N, D_OUT), jnp.float32),
        compiler_params=pltpu.CompilerParams(
            dimension_semantics=("arbitrary",)),
        interpret=interpret,
    )(step_g, step_t, offsets, x_sorted, W1, b1[:, None, :], W2,
      b2[:, None, :])


def _routing(plane_idx, num_planes, num_tiles):
    n = plane_idx.shape[0]
    perm = jnp.argsort(plane_idx).astype(jnp.int32)
    inv_perm = jnp.argsort(perm).astype(jnp.int32)
    counts = jnp.bincount(plane_idx, length=num_planes)
    offsets = jnp.concatenate(
        [jnp.zeros((1,), jnp.int32),
         jnp.cumsum(counts).astype(jnp.int32)])
    t_start = offsets[:num_planes] // _B
    t_end = jnp.maximum(offsets[1:] - 1, 0) // _B
    nsteps = jnp.where(counts > 0, t_end - t_start + 1, 0).astype(jnp.int32)
    cum = jnp.cumsum(nsteps)
    first = cum - nsteps
    total = cum[-1]
    S = num_tiles + num_planes - 1
    s_idx = jnp.arange(S, dtype=jnp.int32)
    g_raw = jnp.searchsorted(cum, s_idx, side="right").astype(jnp.int32)
    real = s_idx < total
    g_last = jnp.searchsorted(cum, total - 1, side="right").astype(jnp.int32)
    g = jnp.where(real, jnp.minimum(g_raw, num_planes - 1), g_last)
    t = jnp.where(real, t_start[g] + s_idx - first[g],
                  num_tiles - 1).astype(jnp.int32)
    return perm, inv_perm, offsets, g, t


def kernel(x, plane_idx, W1, b1, W2, b2):
    N = x.shape[0]
    Lp = W1.shape[0]
    T = N // _B
    perm, inv_perm, offsets, step_g, step_t = _routing(plane_idx, Lp, T)
    x_sorted = jnp.take(x, perm, axis=0)
    out_sorted = _grouped_mlp(x_sorted, offsets, step_g, step_t,
                              W1, b1, W2, b2)
    return jnp.take(out_sorted, inv_perm, axis=0)
